# baseline (speedup 1.0000x reference)
import functools

import jax
import jax.numpy as jnp
from jax.experimental import pallas as pl
from jax.experimental.pallas import tpu as pltpu


def _make_body(tm):
    def body(x_ref, a0_ref, a1_ref, ws_ref, w0_ref, w1_ref, b_ref, o_ref,
             y0_s, y1_s, s_s):
        i = pl.program_id(0)

        @pl.when(i == 0)
        def _prologue():
            x = x_ref[...]
            y0_s[...] = jnp.dot(x, w0_ref[...], preferred_element_type=jnp.float32)
            y1_s[...] = jnp.dot(x, w1_ref[...], preferred_element_type=jnp.float32)
            s_s[...] = jnp.dot(x, ws_ref[...],
                               preferred_element_type=jnp.float32) + b_ref[...]

        acc = s_s[pl.ds(i * tm, tm), :]
        acc += jnp.dot(a0_ref[...], y0_s[...], preferred_element_type=jnp.float32)
        acc += jnp.dot(a1_ref[...], y1_s[...], preferred_element_type=jnp.float32)
        o_ref[...] = jnp.maximum(acc, 0.0)

    return body


@functools.partial(jax.jit, static_argnames=("tm",))
def _rgcn(x, a0, a1, ws, w0, w1, b, tm=256):
    n, f = x.shape
    u = ws.shape[1]
    ni = n // tm
    b2 = b.reshape(1, u)

    out = pl.pallas_call(
        _make_body(tm),
        grid=(ni,),
        in_specs=[
            pl.BlockSpec((n, f), lambda i: (0, 0)),
            pl.BlockSpec((tm, n), lambda i: (i, 0)),
            pl.BlockSpec((tm, n), lambda i: (i, 0)),
            pl.BlockSpec((f, u), lambda i: (0, 0)),
            pl.BlockSpec((f, u), lambda i: (0, 0)),
            pl.BlockSpec((f, u), lambda i: (0, 0)),
            pl.BlockSpec((1, u), lambda i: (0, 0)),
        ],
        out_specs=pl.BlockSpec((tm, u), lambda i: (i, 0)),
        out_shape=jax.ShapeDtypeStruct((n, u), jnp.float32),
        scratch_shapes=[
            pltpu.VMEM((n, u), jnp.float32),
            pltpu.VMEM((n, u), jnp.float32),
            pltpu.VMEM((n, u), jnp.float32),
        ],
        compiler_params=pltpu.CompilerParams(
            dimension_semantics=("arbitrary",)),
    )(x, a0, a1, ws, w0, w1, b2)
    return out


def kernel(features, A_0, A_1, self_kernel, rel_kernel_0, rel_kernel_1, bias):
    x = features[0]
    out = _rgcn(x, A_0, A_1, self_kernel, rel_kernel_0, rel_kernel_1, bias)
    return out[None, ...]
